# Initial kernel scaffold; baseline (speedup 1.0000x reference)
import jax
import jax.numpy as jnp
from jax.experimental import pallas as pl


def kernel(x):
    raise NotImplementedError("write your pallas kernel here")


# speedup vs baseline: 1.0348x; 1.0348x over previous
import functools

import jax
import jax.numpy as jnp
from jax import lax
from jax.experimental import pallas as pl
from jax.experimental.pallas import tpu as pltpu, tpu_sc as plsc

B = 16384
N = 1000
NC = 2
NS = 16
L = 16
NW = NC * NS
ROWS_PER_W = B // NW
CH = L
NCHUNK = ROWS_PER_W // CH
BUF = CH * N


def _onehot_body(x_hbm, out_hbm, idx_v, buf0, buf1, sem0, sem1):
    wid = lax.axis_index("s") * NC + lax.axis_index("c")
    row0 = wid * ROWS_PER_W
    out_base = row0 * N

    pltpu.sync_copy(x_hbm.at[pl.ds(row0, ROWS_PER_W)], idx_v)

    zeros16 = jnp.zeros((L,), jnp.float32)

    def zero_step(i, carry):
        buf0[pl.ds(i * L, L)] = zeros16
        buf1[pl.ds(i * L, L)] = zeros16
        return carry

    lax.fori_loop(0, BUF // L, zero_step, 0)

    ones16 = jnp.full((L,), 1.0, jnp.float32)
    fi_base = lax.iota(jnp.int32, L) * N

    bufs = (buf0, buf1)
    sems = (sem0, sem1)
    copies = [None, None]
    prev_fi = [None, None]
    for c in range(NCHUNK):
        b = c % 2
        buf = bufs[b]
        if copies[b] is not None:
            copies[b].wait()
            plsc.store_scatter(buf, [prev_fi[b]], zeros16)
        xv = idx_v[pl.ds(c * L, L)]
        fi = fi_base + xv
        plsc.store_scatter(buf, [fi], ones16)
        copies[b] = pltpu.async_copy(
            buf, out_hbm.at[pl.ds(out_base + c * BUF, BUF)], sems[b]
        )
        prev_fi[b] = fi
    copies[0].wait()
    copies[1].wait()


@functools.partial(jax.jit, static_argnames=())
def _onehot_flat(x):
    mesh = plsc.VectorSubcoreMesh(
        core_axis_name="c", subcore_axis_name="s", num_cores=NC, num_subcores=NS
    )
    return pl.kernel(
        _onehot_body,
        out_type=jax.ShapeDtypeStruct((B * N,), jnp.float32),
        mesh=mesh,
        scratch_types=[
            pltpu.VMEM((ROWS_PER_W,), jnp.int32),
            pltpu.VMEM((BUF,), jnp.float32),
            pltpu.VMEM((BUF,), jnp.float32),
            pltpu.SemaphoreType.DMA,
            pltpu.SemaphoreType.DMA,
        ],
        compiler_params=pltpu.CompilerParams(needs_layout_passes=False),
        name="onehot_sc",
    )(x)


def kernel(x):
    flat = _onehot_flat(x.astype(jnp.int32))
    return flat.reshape(B, N)


# speedup vs baseline: 1.6591x; 1.6034x over previous
import functools

import jax
import jax.numpy as jnp
from jax import lax
from jax.experimental import pallas as pl
from jax.experimental.pallas import tpu as pltpu, tpu_sc as plsc

B = 16384
N = 1000
NC = 2
NS = 16
L = 16
NW = NC * NS
ROWS_PER_W = B // NW
CH = L
NCHUNK = ROWS_PER_W // CH


def _onehot_body(x_hbm, out_hbm, idx_v, buf0, buf1, sem0, sem1):
    wid = lax.axis_index("s") * NC + lax.axis_index("c")
    row0 = wid * ROWS_PER_W

    pltpu.sync_copy(x_hbm.at[pl.ds(row0, ROWS_PER_W)], idx_v)

    zeros16 = jnp.zeros((L,), jnp.float32)
    n_full = N // L

    def zero_row(r, carry):
        def zero_col(c, carry2):
            buf0[r, pl.ds(c * L, L)] = zeros16
            buf1[r, pl.ds(c * L, L)] = zeros16
            return carry2

        lax.fori_loop(0, n_full, zero_col, 0)
        buf0[r, pl.ds(N - L, L)] = zeros16
        buf1[r, pl.ds(N - L, L)] = zeros16
        return carry

    lax.fori_loop(0, CH, zero_row, 0)

    ones16 = jnp.full((L,), 1.0, jnp.float32)
    rows16 = lax.iota(jnp.int32, L)

    bufs = (buf0, buf1)
    sems = (sem0, sem1)
    copies = [None, None]
    prev_xv = [None, None]
    for c in range(NCHUNK):
        b = c % 2
        buf = bufs[b]
        if copies[b] is not None:
            copies[b].wait()
            plsc.store_scatter(buf, [rows16, prev_xv[b]], zeros16)
        xv = idx_v[pl.ds(c * L, L)]
        plsc.store_scatter(buf, [rows16, xv], ones16)
        copies[b] = pltpu.async_copy(
            buf, out_hbm.at[pl.ds(row0 + c * CH, CH)], sems[b]
        )
        prev_xv[b] = xv
    copies[0].wait()
    copies[1].wait()


@functools.partial(jax.jit, static_argnames=())
def _onehot(x):
    mesh = plsc.VectorSubcoreMesh(
        core_axis_name="c", subcore_axis_name="s", num_cores=NC, num_subcores=NS
    )
    return pl.kernel(
        _onehot_body,
        out_type=jax.ShapeDtypeStruct((B, N), jnp.float32),
        mesh=mesh,
        scratch_types=[
            pltpu.VMEM((ROWS_PER_W,), jnp.int32),
            pltpu.VMEM((CH, N), jnp.float32),
            pltpu.VMEM((CH, N), jnp.float32),
            pltpu.SemaphoreType.DMA,
            pltpu.SemaphoreType.DMA,
        ],
        compiler_params=pltpu.CompilerParams(needs_layout_passes=False),
        name="onehot_sc",
    )(x)


def kernel(x):
    return _onehot(x.astype(jnp.int32))


# speedup vs baseline: 3.8133x; 2.2984x over previous
import functools

import jax
import jax.numpy as jnp
from jax import lax
from jax.experimental import pallas as pl
from jax.experimental.pallas import tpu as pltpu, tpu_sc as plsc

B = 16384
N = 1000
NC = 2
NS = 16
L = 16
NW = NC * NS
COLS_PER_W = B // NW
NGRP = COLS_PER_W // L
RH = 40
NCHUNK = N // RH
NPAIR = (NCHUNK - 1) // 2


def _onehot_t_body(x_hbm, out_hbm, idx_v, buf0, buf1, sem0, sem1):
    wid = lax.axis_index("s") * NC + lax.axis_index("c")
    col0 = wid * COLS_PER_W

    pltpu.sync_copy(x_hbm.at[pl.ds(col0, COLS_PER_W)], idx_v)

    zeros16 = jnp.zeros((L,), jnp.float32)

    def zero_row(r, carry):
        for g in range(NGRP):
            buf0[r, pl.ds(g * L, L)] = zeros16
            buf1[r, pl.ds(g * L, L)] = zeros16
        return carry

    lax.fori_loop(0, RH, zero_row, 0)

    ones16 = jnp.full((L,), 1.0, jnp.float32)
    lanes = lax.iota(jnp.int32, L)

    def scatter_chunk(buf, r0, val):
        for g in range(NGRP):
            xv = idx_v[pl.ds(g * L, L)]
            mask = (xv >= r0) & (xv < r0 + RH)
            plsc.store_scatter(buf, [xv - r0, lanes + g * L], val, mask=mask)

    def issue(buf, r0, sem):
        return pltpu.async_copy(
            buf, out_hbm.at[pl.ds(r0, RH), pl.ds(col0, COLS_PER_W)], sem
        )

    def do_chunk(buf, sem, c, is_first):
        r0 = c * RH

        def reuse():
            pltpu.make_async_copy(
                buf, out_hbm.at[pl.ds(r0, RH), pl.ds(col0, COLS_PER_W)], sem
            ).wait()
            scatter_chunk(buf, r0 - 2 * RH, zeros16)

        if is_first is None:
            reuse()
        else:
            pl.when(jnp.logical_not(is_first))(reuse)
        scatter_chunk(buf, r0, ones16)
        return issue(buf, r0, sem)

    def pair(t, carry):
        do_chunk(buf0, sem0, 2 * t, is_first=t == 0)
        do_chunk(buf1, sem1, 2 * t + 1, is_first=t == 0)
        return carry

    lax.fori_loop(0, NPAIR, pair, 0, unroll=False)

    tail = NCHUNK - 1
    do_chunk(buf0, sem0, tail, is_first=None)
    pltpu.make_async_copy(
        buf1,
        out_hbm.at[pl.ds((tail - 1) * RH, RH), pl.ds(col0, COLS_PER_W)],
        sem1,
    ).wait()
    pltpu.make_async_copy(
        buf0,
        out_hbm.at[pl.ds(tail * RH, RH), pl.ds(col0, COLS_PER_W)],
        sem0,
    ).wait()


@functools.partial(jax.jit, static_argnames=())
def _onehot(x):
    mesh = plsc.VectorSubcoreMesh(
        core_axis_name="c", subcore_axis_name="s", num_cores=NC, num_subcores=NS
    )
    out_t = pl.kernel(
        _onehot_t_body,
        out_type=jax.ShapeDtypeStruct((N, B), jnp.float32),
        mesh=mesh,
        scratch_types=[
            pltpu.VMEM((COLS_PER_W,), jnp.int32),
            pltpu.VMEM((RH, COLS_PER_W), jnp.float32),
            pltpu.VMEM((RH, COLS_PER_W), jnp.float32),
            pltpu.SemaphoreType.DMA,
            pltpu.SemaphoreType.DMA,
        ],
        compiler_params=pltpu.CompilerParams(needs_layout_passes=False),
        name="onehot_sc_t",
    )(x)
    return out_t.T


def kernel(x):
    return _onehot(x.astype(jnp.int32))
